# Initial kernel scaffold; baseline (speedup 1.0000x reference)
import jax
import jax.numpy as jnp
from jax.experimental import pallas as pl


def kernel(x):
    raise NotImplementedError("write your pallas kernel here")


# speedup vs baseline: 15.6960x; 15.6960x over previous
import jax
import jax.numpy as jnp
from jax.experimental import pallas as pl
from jax.experimental.pallas import tpu as pltpu

N = 65536
D = 64
K = 512
B = 2048
NB = N // B
PASSES = 5


def _fused_kernel(x_ref, out_ref, centers, stats, c2, acc):
    p = pl.program_id(0)
    j = pl.program_id(1)
    xb = x_ref[...]

    @pl.when(j == 0)
    def _prologue():
        @pl.when(p == 0)
        def _():
            centers[...] = xb[:K, :]

        @pl.when(p > 0)
        def _():
            st = stats[...]
            cnt = st[:, D:D + 1]
            new_c = st[:, :D] / jnp.maximum(cnt, 1.0)
            centers[...] = jnp.where(cnt > 0.0, new_c, centers[...])

        c = centers[...]
        c2[...] = jax.lax.dot_general(
            jnp.ones((1, D), jnp.float32), c * c,
            (((1,), (1,)), ((), ())), preferred_element_type=jnp.float32)
        stats[...] = jnp.zeros_like(stats)
        acc[...] = jnp.zeros_like(acc)

    c = centers[...]
    xc = jax.lax.dot_general(
        xb, c, (((1,), (1,)), ((), ())),
        preferred_element_type=jnp.float32)
    score = c2[...] - 2.0 * xc
    m = jnp.min(score, axis=1, keepdims=True)
    col = jax.lax.broadcasted_iota(jnp.int32, (B, K), 1)
    cid = jnp.min(jnp.where(score == m, col, K), axis=1, keepdims=True)

    onehot = (col == cid).astype(jnp.float32)
    xe = jnp.concatenate([xb, jnp.ones((B, 1), jnp.float32)], axis=1)
    stats[...] += jax.lax.dot_general(
        onehot, xe, (((0,), (0,)), ((), ())),
        preferred_element_type=jnp.float32)

    @pl.when(p == PASSES - 1)
    def _final():
        rowmax = jnp.max(xc, axis=1, keepdims=True)
        lse = jnp.log(jnp.sum(jnp.exp(xc - rowmax), axis=1, keepdims=True)) + rowmax
        acc[...] += jnp.sum(lse, axis=0, keepdims=True)

        @pl.when(j == NB - 1)
        def _epilogue():
            st = stats[...]
            occ = (st[:, D:D + 1] > 0.0).astype(jnp.float32)
            mm = jax.lax.broadcasted_iota(jnp.int32, (K, K), 1)
            kk = jax.lax.broadcasted_iota(jnp.int32, (K, K), 0)
            lt = (mm < kk).astype(jnp.float32)
            prefix = jax.lax.dot_general(
                lt, occ, (((1,), (0,)), ((), ())),
                preferred_element_type=jnp.float32)
            sel = (mm == prefix.astype(jnp.int32)).astype(jnp.float32)
            gathered = jax.lax.dot_general(
                sel, centers[...], (((1,), (0,)), ((), ())),
                preferred_element_type=jnp.float32)
            picked_sum = jnp.sum(st[:, :D] * gathered)
            out_ref[...] = (acc[...] - picked_sum) * (1.0 / N)


def kernel(x):
    loss2d = pl.pallas_call(
        _fused_kernel,
        grid=(PASSES, NB),
        in_specs=[pl.BlockSpec((B, D), lambda p, j: (j, 0))],
        out_specs=pl.BlockSpec((1, 1), lambda p, j: (0, 0)),
        out_shape=jax.ShapeDtypeStruct((1, 1), jnp.float32),
        scratch_shapes=[
            pltpu.VMEM((K, D), jnp.float32),
            pltpu.VMEM((K, D + 1), jnp.float32),
            pltpu.VMEM((1, K), jnp.float32),
            pltpu.VMEM((1, 1), jnp.float32),
        ],
        compiler_params=pltpu.CompilerParams(
            dimension_semantics=("arbitrary", "arbitrary")),
    )(x)
    return loss2d[0, 0]


# speedup vs baseline: 18.0628x; 1.1508x over previous
import jax
import jax.numpy as jnp
from jax.experimental import pallas as pl
from jax.experimental.pallas import tpu as pltpu

N = 65536
D = 64
K = 512
B = 2048
NB = N // B
PASSES = 5


def _fused_kernel(x_ref, out_ref, ct, stats, c2, acc):
    p = pl.program_id(0)
    j = pl.program_id(1)
    xb = x_ref[...]

    @pl.when(j == 0)
    def _prologue():
        @pl.when(p == 0)
        def _():
            ct[...] = jax.lax.transpose(xb[:K, :], (1, 0))

        @pl.when(p > 0)
        def _():
            st = stats[...]
            cnt = st[D:D + 1, :]
            new_ct = st[:D, :] / jnp.maximum(cnt, 1.0)
            ct[...] = jnp.where(cnt > 0.0, new_ct, ct[...])

        c = ct[...]
        c2[...] = jax.lax.dot_general(
            jnp.ones((1, D), jnp.float32), c * c,
            (((1,), (0,)), ((), ())), preferred_element_type=jnp.float32)
        stats[...] = jnp.zeros_like(stats)
        acc[...] = jnp.zeros_like(acc)

    xc = jnp.dot(xb, ct[...], preferred_element_type=jnp.float32)
    score = c2[...] - 2.0 * xc
    m = jnp.min(score, axis=1, keepdims=True)
    col = jax.lax.broadcasted_iota(jnp.int32, (B, K), 1)
    cid = jnp.min(jnp.where(score == m, col, K), axis=1, keepdims=True)

    onehot = (col == cid).astype(jnp.float32)
    xe = jnp.concatenate([xb, jnp.ones((B, 1), jnp.float32)], axis=1)
    stats[...] += jax.lax.dot_general(
        xe, onehot, (((0,), (0,)), ((), ())),
        preferred_element_type=jnp.float32)

    @pl.when(p == PASSES - 1)
    def _final():
        rowmax = jnp.max(xc, axis=1, keepdims=True)
        lse = jnp.log(jnp.sum(jnp.exp(xc - rowmax), axis=1, keepdims=True)) + rowmax
        acc[...] += jnp.sum(lse, axis=0, keepdims=True)

        @pl.when(j == NB - 1)
        def _epilogue():
            st = stats[...]
            occ = (st[D:D + 1, :] > 0.0).astype(jnp.float32)
            mm = jax.lax.broadcasted_iota(jnp.int32, (K, K), 0)
            kk = jax.lax.broadcasted_iota(jnp.int32, (K, K), 1)
            lt = (mm < kk).astype(jnp.float32)
            prefix = jax.lax.dot_general(
                occ, lt, (((1,), (0,)), ((), ())),
                preferred_element_type=jnp.float32)
            sel = (mm == prefix.astype(jnp.int32)).astype(jnp.float32)
            gathered = jax.lax.dot_general(
                ct[...], sel, (((1,), (0,)), ((), ())),
                preferred_element_type=jnp.float32)
            picked_sum = jnp.sum(st[:D, :] * gathered)
            out_ref[...] = (acc[...] - picked_sum) * (1.0 / N)


def kernel(x):
    loss2d = pl.pallas_call(
        _fused_kernel,
        grid=(PASSES, NB),
        in_specs=[pl.BlockSpec((B, D), lambda p, j: (j, 0))],
        out_specs=pl.BlockSpec((1, 1), lambda p, j: (0, 0)),
        out_shape=jax.ShapeDtypeStruct((1, 1), jnp.float32),
        scratch_shapes=[
            pltpu.VMEM((D, K), jnp.float32),
            pltpu.VMEM((D + 1, K), jnp.float32),
            pltpu.VMEM((1, K), jnp.float32),
            pltpu.VMEM((1, 1), jnp.float32),
        ],
        compiler_params=pltpu.CompilerParams(
            dimension_semantics=("arbitrary", "arbitrary")),
    )(x)
    return loss2d[0, 0]


# speedup vs baseline: 20.6634x; 1.1440x over previous
import jax
import jax.numpy as jnp
from jax.experimental import pallas as pl
from jax.experimental.pallas import tpu as pltpu

N = 65536
D = 64
K = 512
B = 4096
NB = N // B
PASSES = 5


def _fused_kernel(x_ref, out_ref, ct, cte, stats, acc):
    p = pl.program_id(0)
    j = pl.program_id(1)
    xb = x_ref[...]

    @pl.when(j == 0)
    def _prologue():
        @pl.when(p == 0)
        def _():
            ct[...] = jax.lax.transpose(xb[:K, :], (1, 0))

        @pl.when(p > 0)
        def _():
            st = stats[...]
            cnt = st[D:D + 1, :]
            new_ct = st[:D, :] / jnp.maximum(cnt, 1.0)
            ct[...] = jnp.where(cnt > 0.0, new_ct, ct[...])

        c = ct[...]
        cte[:D, :] = -2.0 * c
        cte[D:D + 1, :] = jax.lax.dot_general(
            jnp.ones((1, D), jnp.float32), c * c,
            (((1,), (0,)), ((), ())), preferred_element_type=jnp.float32)
        stats[...] = jnp.zeros_like(stats)
        acc[...] = jnp.zeros_like(acc)

    xe = jnp.concatenate([xb, jnp.ones((B, 1), jnp.float32)], axis=1)
    score = jnp.dot(xe, cte[...], preferred_element_type=jnp.float32)
    m = jnp.min(score, axis=1, keepdims=True)
    col = jax.lax.broadcasted_iota(jnp.int32, (B, K), 1)
    cid = jnp.min(jnp.where(score == m, col, K), axis=1, keepdims=True)

    onehot = (col == cid).astype(jnp.float32)
    stats[...] += jax.lax.dot_general(
        xe, onehot, (((0,), (0,)), ((), ())),
        preferred_element_type=jnp.float32)

    @pl.when(p == PASSES - 1)
    def _final():
        xc = 0.5 * (cte[D:D + 1, :] - score)
        rowmax = jnp.max(xc, axis=1, keepdims=True)
        lse = jnp.log(jnp.sum(jnp.exp(xc - rowmax), axis=1, keepdims=True)) + rowmax
        acc[...] += jnp.sum(lse, axis=0, keepdims=True)

        @pl.when(j == NB - 1)
        def _epilogue():
            st = stats[...]
            occ = (st[D:D + 1, :] > 0.0).astype(jnp.float32)
            mm = jax.lax.broadcasted_iota(jnp.int32, (K, K), 0)
            kk = jax.lax.broadcasted_iota(jnp.int32, (K, K), 1)
            lt = (mm < kk).astype(jnp.float32)
            prefix = jax.lax.dot_general(
                occ, lt, (((1,), (0,)), ((), ())),
                preferred_element_type=jnp.float32)
            sel = (mm == prefix.astype(jnp.int32)).astype(jnp.float32)
            gathered = jax.lax.dot_general(
                ct[...], sel, (((1,), (0,)), ((), ())),
                preferred_element_type=jnp.float32)
            picked_sum = jnp.sum(st[:D, :] * gathered)
            out_ref[...] = (acc[...] - picked_sum) * (1.0 / N)


def kernel(x):
    loss2d = pl.pallas_call(
        _fused_kernel,
        grid=(PASSES, NB),
        in_specs=[pl.BlockSpec((B, D), lambda p, j: (j, 0))],
        out_specs=pl.BlockSpec((1, 1), lambda p, j: (0, 0)),
        out_shape=jax.ShapeDtypeStruct((1, 1), jnp.float32),
        scratch_shapes=[
            pltpu.VMEM((D, K), jnp.float32),
            pltpu.VMEM((D + 1, K), jnp.float32),
            pltpu.VMEM((D + 1, K), jnp.float32),
            pltpu.VMEM((1, 1), jnp.float32),
        ],
        compiler_params=pltpu.CompilerParams(
            dimension_semantics=("arbitrary", "arbitrary")),
    )(x)
    return loss2d[0, 0]
